# baseline (speedup 1.0000x reference)
import jax
import jax.numpy as jnp
from jax import lax
from jax.experimental import pallas as pl
from jax.experimental.pallas import tpu as pltpu
from jax.experimental.pallas import tpu_sc as plsc

B = 16384
EMB = 256
HID = 512
F32 = jnp.float32
BF16 = jnp.bfloat16

NC, NS = 2, 16
NW = NC * NS
BPW = B // NW
CHUNK = 64
NCHUNK = BPW // CHUNK


def _gelu(x):
    return 0.5 * x * (1.0 + lax.erf(x * 0.7071067811865476))


def _pack_words(x):
    lo = lax.bitcast_convert_type(x[:, 0:256].astype(BF16), jnp.uint16)
    hi = lax.bitcast_convert_type(x[:, 256:512].astype(BF16), jnp.uint16)
    return (hi.astype(jnp.int32) << 16) | lo.astype(jnp.int32)


def _fold_body(ct, rt, lt, fW1, cW2, cb2, fb1, wc, wrl, m):
    bvec = jnp.dot(cb2[...].reshape(1, 64), fW1[512:576, :],
                   preferred_element_type=F32) + fb1[...].reshape(1, HID)
    wc[...] = _pack_words(
        jnp.dot(ct[...], fW1[0:256, :], preferred_element_type=F32) + bvec)
    wr = jnp.dot(rt[...], fW1[256:384, :], preferred_element_type=F32)
    wl = jnp.dot(lt[...], fW1[384:512, :], preferred_element_type=F32)
    wrl[...] = _pack_words(
        (wr.reshape(20, 1, HID) + wl.reshape(1, 100, HID)).reshape(2000, HID))
    m[...] = jnp.dot(cW2[...], fW1[512:576, :], preferred_element_type=F32)


def _fold(ct, rt, lt, fW1, cW2, cb2, fb1):
    return pl.pallas_call(
        _fold_body,
        out_shape=(
            jax.ShapeDtypeStruct((200, HID // 2), jnp.int32),
            jax.ShapeDtypeStruct((2000, HID // 2), jnp.int32),
            jax.ShapeDtypeStruct((64, HID), F32),
        ),
    )(ct, rt, lt, fW1, cW2, cb2, fb1)


W = HID // 2
NSLICE = 2


def _make_sc_gather_body(nrows):
    bpw = nrows // NW
    nchunk = bpw // CHUNK

    def body(wc_hbm, wrl_hbm, cid_hbm, pid_hbm, gc_hbm, grl_hbm,
             cid_v, pid_v, bc0, bc1, br0, br1, sc0, sc1, sr0, sr1):
        wid = lax.axis_index("s") * NC + lax.axis_index("c")
        base = wid * bpw
        pltpu.sync_copy(cid_hbm.at[pl.ds(base, bpw)], cid_v)
        pltpu.sync_copy(pid_hbm.at[pl.ds(base, bpw)], pid_v)

        bcs, brs = (bc0, bc1), (br0, br1)
        scs, srs = (sc0, sc1), (sr0, sr1)
        gdesc = [None] * nchunk

        def emit_out(i):
            q = i % 2
            oo = i * CHUNK
            gdesc[i][0].wait()
            pltpu.sync_copy(bcs[q], gc_hbm.at[pl.ds(base + oo, CHUNK)])
            gdesc[i][1].wait()
            pltpu.sync_copy(brs[q], grl_hbm.at[pl.ds(base + oo, CHUNK)])

        for i in range(nchunk):
            o = i * CHUNK
            p = i % 2
            gdesc[i] = (
                pltpu.async_copy(wc_hbm.at[cid_v.at[pl.ds(o, CHUNK)]],
                                 bcs[p], scs[p]),
                pltpu.async_copy(wrl_hbm.at[pid_v.at[pl.ds(o, CHUNK)]],
                                 brs[p], srs[p]),
            )
            if i >= 1:
                emit_out(i - 1)
        emit_out(nchunk - 1)

    return body, bpw


def _sc_gather(wc, wrl, cid, pid):
    nrows = cid.shape[0]
    body, bpw = _make_sc_gather_body(nrows)
    mesh = plsc.VectorSubcoreMesh(core_axis_name="c", subcore_axis_name="s")
    return pl.kernel(
        body,
        out_type=(jax.ShapeDtypeStruct((nrows, W), jnp.int32),
                  jax.ShapeDtypeStruct((nrows, W), jnp.int32)),
        mesh=mesh,
        scratch_types=[
            pltpu.VMEM((bpw,), jnp.int32),
            pltpu.VMEM((bpw,), jnp.int32),
            pltpu.VMEM((CHUNK, W), jnp.int32),
            pltpu.VMEM((CHUNK, W), jnp.int32),
            pltpu.VMEM((CHUNK, W), jnp.int32),
            pltpu.VMEM((CHUNK, W), jnp.int32),
            pltpu.SemaphoreType.DMA,
            pltpu.SemaphoreType.DMA,
            pltpu.SemaphoreType.DMA,
            pltpu.SemaphoreType.DMA,
        ],
    )(wc, wrl, cid, pid)


BLK = 1024


def _unpack_lo_hi(w):
    lo = lax.bitcast_convert_type(w << 16, F32)
    hi = lax.bitcast_convert_type(w & jnp.int32(-65536), F32)
    return lo, hi


def _main_body(gc, grl, ch, cW1, cb1, m, fW2, fb2, o):
    h = _gelu(jnp.dot(ch[...], cW1[...], preferred_element_type=F32) + cb1[...])
    c_lo, c_hi = _unpack_lo_hi(gc[...])
    r_lo, r_hi = _unpack_lo_hi(grl[...])
    t_lo = jnp.dot(h, m[:, 0:256], preferred_element_type=F32)
    t_hi = jnp.dot(h, m[:, 256:512], preferred_element_type=F32)
    h2_lo = _gelu(c_lo + r_lo + t_lo)
    h2_hi = _gelu(c_hi + r_hi + t_hi)
    o[...] = (jnp.dot(h2_lo, fW2[0:256, :], preferred_element_type=F32)
              + jnp.dot(h2_hi, fW2[256:512, :], preferred_element_type=F32)
              + fb2[...])


def _tc_main(gc, grl, ch, cW1, cb1, m, fW2, fb2):
    nrows = gc.shape[0]
    grid = (nrows // BLK,)
    return pl.pallas_call(
        _main_body,
        grid=grid,
        in_specs=[
            pl.BlockSpec((BLK, HID // 2), lambda i: (i, 0)),
            pl.BlockSpec((BLK, HID // 2), lambda i: (i, 0)),
            pl.BlockSpec((BLK, 16), lambda i: (i, 0)),
            pl.BlockSpec((16, 64), lambda i: (0, 0)),
            pl.BlockSpec((64,), lambda i: (0,)),
            pl.BlockSpec((64, HID), lambda i: (0, 0)),
            pl.BlockSpec((HID, HID), lambda i: (0, 0)),
            pl.BlockSpec((HID,), lambda i: (0,)),
        ],
        out_specs=pl.BlockSpec((BLK, HID), lambda i: (i, 0)),
        out_shape=jax.ShapeDtypeStruct((nrows, HID), F32),
    )(gc, grl, ch, cW1, cb1, m, fW2, fb2)


def kernel(country_ids, region_ids, language_ids, characteristics,
           country_table, region_table, lang_table,
           cW1, cb1, cW2, cb2, fW1, fb1, fW2, fb2):
    cid = country_ids.astype(jnp.int32)
    pid = region_ids.astype(jnp.int32) * 100 + language_ids.astype(jnp.int32)
    wc, wrl, m = _fold(country_table, region_table, lang_table,
                       fW1, cW2, cb2, fb1)
    bs = B // NSLICE
    gs = [_sc_gather(wc, wrl, cid[s * bs:(s + 1) * bs],
                     pid[s * bs:(s + 1) * bs]) for s in range(NSLICE)]
    outs = [_tc_main(gc, grl, characteristics[s * bs:(s + 1) * bs],
                     cW1, cb1, m, fW2, fb2)
            for s, (gc, grl) in enumerate(gs)]
    if NSLICE == 1:
        return outs[0]
    return jnp.concatenate(outs, axis=0)


# speedup vs baseline: 1.2193x; 1.2193x over previous
import jax
import jax.numpy as jnp
from jax import lax
from jax.experimental import pallas as pl
from jax.experimental.pallas import tpu as pltpu
from jax.experimental.pallas import tpu_sc as plsc

B = 16384
EMB = 256
HID = 512
F32 = jnp.float32
BF16 = jnp.bfloat16

NC, NS = 2, 16
NW = NC * NS
BPW = B // NW
CHUNK = 64
NCHUNK = BPW // CHUNK


def _gelu(x):
    return 0.5 * x * (1.0 + lax.erf(x * 0.7071067811865476))


def _pack_words(x):
    lo = lax.bitcast_convert_type(x[:, 0:256].astype(BF16), jnp.uint16)
    hi = lax.bitcast_convert_type(x[:, 256:512].astype(BF16), jnp.uint16)
    return (hi.astype(jnp.int32) << 16) | lo.astype(jnp.int32)


def _fold_body(ct, rt, lt, fW1, cW2, cb2, fb1, fW2, wc, wrl, m, w2b):
    bvec = jnp.dot(cb2[...].reshape(1, 64), fW1[512:576, :],
                   preferred_element_type=F32) + fb1[...].reshape(1, HID)
    wc[...] = _pack_words(
        jnp.dot(ct[...], fW1[0:256, :], preferred_element_type=F32) + bvec)
    wr = jnp.dot(rt[...], fW1[256:384, :], preferred_element_type=F32)
    wl = jnp.dot(lt[...], fW1[384:512, :], preferred_element_type=F32)
    wrl[...] = _pack_words(
        (wr.reshape(20, 1, HID) + wl.reshape(1, 100, HID)).reshape(2000, HID))
    m[...] = jnp.dot(cW2[...], fW1[512:576, :],
                     preferred_element_type=F32).astype(BF16)
    w2b[...] = fW2[...].astype(BF16)


def _fold(ct, rt, lt, fW1, cW2, cb2, fb1, fW2):
    return pl.pallas_call(
        _fold_body,
        out_shape=(
            jax.ShapeDtypeStruct((200, HID // 2), jnp.int32),
            jax.ShapeDtypeStruct((2000, HID // 2), jnp.int32),
            jax.ShapeDtypeStruct((64, HID), BF16),
            jax.ShapeDtypeStruct((HID, HID), BF16),
        ),
    )(ct, rt, lt, fW1, cW2, cb2, fb1, fW2)


W = HID // 2
NSLICE = 1


def _make_sc_gather_body(nrows):
    bpw = nrows // NW
    nchunk = bpw // CHUNK

    def body(wc_hbm, wrl_hbm, cid_hbm, pid_hbm, gc_hbm, grl_hbm,
             cid_v, pid_v, bc0, bc1, br0, br1, sc0, sc1, sr0, sr1):
        wid = lax.axis_index("s") * NC + lax.axis_index("c")
        base = wid * bpw
        pltpu.sync_copy(cid_hbm.at[pl.ds(base, bpw)], cid_v)
        pltpu.sync_copy(pid_hbm.at[pl.ds(base, bpw)], pid_v)

        bcs, brs = (bc0, bc1), (br0, br1)
        scs, srs = (sc0, sc1), (sr0, sr1)
        gdesc = [None] * nchunk

        def emit_out(i):
            q = i % 2
            oo = i * CHUNK
            gdesc[i][0].wait()
            pltpu.sync_copy(bcs[q], gc_hbm.at[pl.ds(base + oo, CHUNK)])
            gdesc[i][1].wait()
            pltpu.sync_copy(brs[q], grl_hbm.at[pl.ds(base + oo, CHUNK)])

        for i in range(nchunk):
            o = i * CHUNK
            p = i % 2
            gdesc[i] = (
                pltpu.async_copy(wc_hbm.at[cid_v.at[pl.ds(o, CHUNK)]],
                                 bcs[p], scs[p]),
                pltpu.async_copy(wrl_hbm.at[pid_v.at[pl.ds(o, CHUNK)]],
                                 brs[p], srs[p]),
            )
            if i >= 1:
                emit_out(i - 1)
        emit_out(nchunk - 1)

    return body, bpw


def _sc_gather(wc, wrl, cid, pid):
    nrows = cid.shape[0]
    body, bpw = _make_sc_gather_body(nrows)
    mesh = plsc.VectorSubcoreMesh(core_axis_name="c", subcore_axis_name="s")
    return pl.kernel(
        body,
        out_type=(jax.ShapeDtypeStruct((nrows, W), jnp.int32),
                  jax.ShapeDtypeStruct((nrows, W), jnp.int32)),
        mesh=mesh,
        scratch_types=[
            pltpu.VMEM((bpw,), jnp.int32),
            pltpu.VMEM((bpw,), jnp.int32),
            pltpu.VMEM((CHUNK, W), jnp.int32),
            pltpu.VMEM((CHUNK, W), jnp.int32),
            pltpu.VMEM((CHUNK, W), jnp.int32),
            pltpu.VMEM((CHUNK, W), jnp.int32),
            pltpu.SemaphoreType.DMA,
            pltpu.SemaphoreType.DMA,
            pltpu.SemaphoreType.DMA,
            pltpu.SemaphoreType.DMA,
        ],
    )(wc, wrl, cid, pid)


BLK = 1024


def _unpack_lo_hi(w):
    lo = lax.bitcast_convert_type(w << 16, F32)
    hi = lax.bitcast_convert_type(w & jnp.int32(-65536), F32)
    return lo, hi


def _main_body(gc, grl, ch, cW1, cb1, m, fW2, fb2, o):
    h = _gelu(jnp.dot(ch[...], cW1[...], preferred_element_type=F32)
              + cb1[...]).astype(BF16)
    c_lo, c_hi = _unpack_lo_hi(gc[...])
    r_lo, r_hi = _unpack_lo_hi(grl[...])
    t_lo = jnp.dot(h, m[:, 0:256], preferred_element_type=F32)
    t_hi = jnp.dot(h, m[:, 256:512], preferred_element_type=F32)
    h2_lo = _gelu(c_lo + r_lo + t_lo).astype(BF16)
    h2_hi = _gelu(c_hi + r_hi + t_hi).astype(BF16)
    o[...] = (jnp.dot(h2_lo, fW2[0:256, :], preferred_element_type=F32)
              + jnp.dot(h2_hi, fW2[256:512, :], preferred_element_type=F32)
              + fb2[...])


def _tc_main(gc, grl, ch, cW1, cb1, m, fW2, fb2):
    nrows = gc.shape[0]
    grid = (nrows // BLK,)
    return pl.pallas_call(
        _main_body,
        grid=grid,
        in_specs=[
            pl.BlockSpec((BLK, HID // 2), lambda i: (i, 0)),
            pl.BlockSpec((BLK, HID // 2), lambda i: (i, 0)),
            pl.BlockSpec((BLK, 16), lambda i: (i, 0)),
            pl.BlockSpec((16, 64), lambda i: (0, 0)),
            pl.BlockSpec((64,), lambda i: (0,)),
            pl.BlockSpec((64, HID), lambda i: (0, 0)),
            pl.BlockSpec((HID, HID), lambda i: (0, 0)),
            pl.BlockSpec((HID,), lambda i: (0,)),
        ],
        out_specs=pl.BlockSpec((BLK, HID), lambda i: (i, 0)),
        out_shape=jax.ShapeDtypeStruct((nrows, HID), F32),
    )(gc, grl, ch, cW1, cb1, m, fW2, fb2)


def kernel(country_ids, region_ids, language_ids, characteristics,
           country_table, region_table, lang_table,
           cW1, cb1, cW2, cb2, fW1, fb1, fW2, fb2):
    cid = country_ids.astype(jnp.int32)
    pid = region_ids.astype(jnp.int32) * 100 + language_ids.astype(jnp.int32)
    wc, wrl, m, fW2b = _fold(country_table, region_table, lang_table,
                             fW1, cW2, cb2, fb1, fW2)
    bs = B // NSLICE
    gs = [_sc_gather(wc, wrl, cid[s * bs:(s + 1) * bs],
                     pid[s * bs:(s + 1) * bs]) for s in range(NSLICE)]
    outs = [_tc_main(gc, grl, characteristics[s * bs:(s + 1) * bs],
                     cW1, cb1, m, fW2b, fb2)
            for s, (gc, grl) in enumerate(gs)]
    if NSLICE == 1:
        return outs[0]
    return jnp.concatenate(outs, axis=0)


# speedup vs baseline: 1.2761x; 1.0465x over previous
import jax
import jax.numpy as jnp
from jax import lax
from jax.experimental import pallas as pl
from jax.experimental.pallas import tpu as pltpu
from jax.experimental.pallas import tpu_sc as plsc

B = 16384
EMB = 256
HID = 512
F32 = jnp.float32
BF16 = jnp.bfloat16

NC, NS = 2, 16
NW = NC * NS
BPW = B // NW
CHUNK = 64
NCHUNK = BPW // CHUNK


def _gelu(x):
    return 0.5 * x * (1.0 + lax.erf(x * 0.7071067811865476))


def _pack_words(x):
    lo = lax.bitcast_convert_type(x[:, 0:256].astype(BF16), jnp.uint16)
    hi = lax.bitcast_convert_type(x[:, 256:512].astype(BF16), jnp.uint16)
    return (hi.astype(jnp.int32) << 16) | lo.astype(jnp.int32)


def _fold_body(ct, rt, lt, fW1, cW2, cb2, fb1, fW2, wc, wrl, m, w2b):
    bvec = jnp.dot(cb2[...].reshape(1, 64), fW1[512:576, :],
                   preferred_element_type=F32) + fb1[...].reshape(1, HID)
    wc[...] = _pack_words(
        jnp.dot(ct[...], fW1[0:256, :], preferred_element_type=F32) + bvec)
    wr = jnp.dot(rt[...], fW1[256:384, :], preferred_element_type=F32)
    wl = jnp.dot(lt[...], fW1[384:512, :], preferred_element_type=F32)
    wrl[...] = _pack_words(
        (wr.reshape(20, 1, HID) + wl.reshape(1, 100, HID)).reshape(2000, HID))
    m[...] = jnp.dot(cW2[...], fW1[512:576, :],
                     preferred_element_type=F32).astype(BF16)
    w2b[...] = fW2[...].astype(BF16)


def _fold(ct, rt, lt, fW1, cW2, cb2, fb1, fW2):
    return pl.pallas_call(
        _fold_body,
        out_shape=(
            jax.ShapeDtypeStruct((200, HID // 2), jnp.int32),
            jax.ShapeDtypeStruct((2000, HID // 2), jnp.int32),
            jax.ShapeDtypeStruct((64, HID), BF16),
            jax.ShapeDtypeStruct((HID, HID), BF16),
        ),
    )(ct, rt, lt, fW1, cW2, cb2, fb1, fW2)


W = HID // 2
NSLICE = 1


def _make_sc_gather_body(nrows):
    bpw = nrows // NW
    nchunk = bpw // CHUNK

    def body(wc_hbm, wrl_hbm, cid_hbm, pid_hbm, gc_hbm, grl_hbm,
             cid_v, pid_v, bc0, bc1, br0, br1, sc0, sc1, sr0, sr1):
        wid = lax.axis_index("s") * NC + lax.axis_index("c")
        base = wid * bpw
        pltpu.sync_copy(cid_hbm.at[pl.ds(base, bpw)], cid_v)
        pltpu.sync_copy(pid_hbm.at[pl.ds(base, bpw)], pid_v)

        bcs, brs = (bc0, bc1), (br0, br1)
        scs, srs = (sc0, sc1), (sr0, sr1)
        gdesc = [None] * nchunk

        def emit_out(i):
            q = i % 2
            oo = i * CHUNK
            gdesc[i][0].wait()
            pltpu.sync_copy(bcs[q], gc_hbm.at[pl.ds(base + oo, CHUNK)])
            gdesc[i][1].wait()
            pltpu.sync_copy(brs[q], grl_hbm.at[pl.ds(base + oo, CHUNK)])

        for i in range(nchunk):
            o = i * CHUNK
            p = i % 2
            gdesc[i] = (
                pltpu.async_copy(wc_hbm.at[cid_v.at[pl.ds(o, CHUNK)]],
                                 bcs[p], scs[p]),
                pltpu.async_copy(wrl_hbm.at[pid_v.at[pl.ds(o, CHUNK)]],
                                 brs[p], srs[p]),
            )
            if i >= 1:
                emit_out(i - 1)
        emit_out(nchunk - 1)

    return body, bpw


def _sc_gather(wc, wrl, cid, pid):
    nrows = cid.shape[0]
    body, bpw = _make_sc_gather_body(nrows)
    mesh = plsc.VectorSubcoreMesh(core_axis_name="c", subcore_axis_name="s")
    return pl.kernel(
        body,
        out_type=(jax.ShapeDtypeStruct((nrows, W), jnp.int32),
                  jax.ShapeDtypeStruct((nrows, W), jnp.int32)),
        mesh=mesh,
        scratch_types=[
            pltpu.VMEM((bpw,), jnp.int32),
            pltpu.VMEM((bpw,), jnp.int32),
            pltpu.VMEM((CHUNK, W), jnp.int32),
            pltpu.VMEM((CHUNK, W), jnp.int32),
            pltpu.VMEM((CHUNK, W), jnp.int32),
            pltpu.VMEM((CHUNK, W), jnp.int32),
            pltpu.SemaphoreType.DMA,
            pltpu.SemaphoreType.DMA,
            pltpu.SemaphoreType.DMA,
            pltpu.SemaphoreType.DMA,
        ],
    )(wc, wrl, cid, pid)


BLK = 2048


def _unpack_lo_hi(w):
    lo = lax.bitcast_convert_type(w << 16, F32)
    hi = lax.bitcast_convert_type(w & jnp.int32(-65536), F32)
    return lo, hi


def _main_body(gc, grl, ch, cW1, cb1, m, fW2, fb2, o):
    h = _gelu(jnp.dot(ch[...], cW1[...], preferred_element_type=F32)
              + cb1[...]).astype(BF16)
    c_lo, c_hi = _unpack_lo_hi(gc[...])
    r_lo, r_hi = _unpack_lo_hi(grl[...])
    t_lo = jnp.dot(h, m[:, 0:256], preferred_element_type=F32)
    t_hi = jnp.dot(h, m[:, 256:512], preferred_element_type=F32)
    h2_lo = _gelu(c_lo + r_lo + t_lo).astype(BF16)
    h2_hi = _gelu(c_hi + r_hi + t_hi).astype(BF16)
    o[...] = (jnp.dot(h2_lo, fW2[0:256, :], preferred_element_type=F32)
              + jnp.dot(h2_hi, fW2[256:512, :], preferred_element_type=F32)
              + fb2[...])


def _tc_main(gc, grl, ch, cW1, cb1, m, fW2, fb2):
    nrows = gc.shape[0]
    grid = (nrows // BLK,)
    return pl.pallas_call(
        _main_body,
        grid=grid,
        in_specs=[
            pl.BlockSpec((BLK, HID // 2), lambda i: (i, 0)),
            pl.BlockSpec((BLK, HID // 2), lambda i: (i, 0)),
            pl.BlockSpec((BLK, 16), lambda i: (i, 0)),
            pl.BlockSpec((16, 64), lambda i: (0, 0)),
            pl.BlockSpec((64,), lambda i: (0,)),
            pl.BlockSpec((64, HID), lambda i: (0, 0)),
            pl.BlockSpec((HID, HID), lambda i: (0, 0)),
            pl.BlockSpec((HID,), lambda i: (0,)),
        ],
        out_specs=pl.BlockSpec((BLK, HID), lambda i: (i, 0)),
        out_shape=jax.ShapeDtypeStruct((nrows, HID), F32),
    )(gc, grl, ch, cW1, cb1, m, fW2, fb2)


def kernel(country_ids, region_ids, language_ids, characteristics,
           country_table, region_table, lang_table,
           cW1, cb1, cW2, cb2, fW1, fb1, fW2, fb2):
    cid = country_ids.astype(jnp.int32)
    pid = region_ids.astype(jnp.int32) * 100 + language_ids.astype(jnp.int32)
    wc, wrl, m, fW2b = _fold(country_table, region_table, lang_table,
                             fW1, cW2, cb2, fb1, fW2)
    bs = B // NSLICE
    gs = [_sc_gather(wc, wrl, cid[s * bs:(s + 1) * bs],
                     pid[s * bs:(s + 1) * bs]) for s in range(NSLICE)]
    outs = [_tc_main(gc, grl, characteristics[s * bs:(s + 1) * bs],
                     cW1, cb1, m, fW2b, fb2)
            for s, (gc, grl) in enumerate(gs)]
    if NSLICE == 1:
        return outs[0]
    return jnp.concatenate(outs, axis=0)
